# baseline (speedup 1.0000x reference)
import functools

import numpy as np

import jax
import jax.numpy as jnp
from jax import lax
from jax.experimental import pallas as pl
from jax.experimental.pallas import tpu as pltpu
from jax.experimental.pallas import tpu_sc as plsc

ROWS = 16384
NW = 32
ROWS_PER = ROWS // NW
IN_PER = ROWS_PER * 2
OUT_PER = ROWS_PER * 3
L = 16

_OFFS = np.arange(48)
_COL = (_OFFS % 3).astype(np.int32)
_COL2 = (_COL * 2).astype(np.int32)
_ROW2 = ((_OFFS // 3) * 2).astype(np.int32)


def _sc_body(x_hbm, w_hbm, b_hbm, emb_hbm, col_hbm, col2_hbm, row2_hbm,
             out_hbm, xv, ov, wv, bv, ev, cv, colv, col2v, row2v):
    nc = 2
    wid = lax.axis_index("s") * nc + lax.axis_index("c")

    pltpu.sync_copy(x_hbm.at[pl.ds(wid * IN_PER, IN_PER)], xv)
    pltpu.sync_copy(w_hbm, wv.at[pl.ds(0, 6)])
    pltpu.sync_copy(b_hbm, bv.at[pl.ds(0, 3)])
    pltpu.sync_copy(emb_hbm.at[pl.ds(0, 3)], ev.at[pl.ds(0, 3)])
    pltpu.sync_copy(col_hbm, colv.at[pl.ds(0, 48)])
    pltpu.sync_copy(col2_hbm, col2v.at[pl.ds(0, 48)])
    pltpu.sync_copy(row2_hbm, row2v.at[pl.ds(0, 48)])

    cv[pl.ds(0, 16)] = bv[pl.ds(0, 16)] + ev[pl.ds(0, 16)]

    was, wbs, cs, rows2 = [], [], [], []
    for p in range(3):
        col = colv[pl.ds(16 * p, 16)]
        col2 = col2v[pl.ds(16 * p, 16)]
        was.append(plsc.load_gather(wv, [col2]))
        wbs.append(plsc.load_gather(wv, [col2 + 1]))
        cs.append(plsc.load_gather(cv, [col]))
        rows2.append(row2v[pl.ds(16 * p, 16)])

    for i in range(ROWS_PER // 16):
        for p in range(3):
            idxa = rows2[p] + (32 * i)
            a = plsc.load_gather(xv, [idxa])
            bcomp = plsc.load_gather(xv, [idxa + 1])
            ov[pl.ds(48 * i + 16 * p, 16)] = a * was[p] + bcomp * wbs[p] + cs[p]

    pltpu.sync_copy(ov, out_hbm.at[pl.ds(wid * OUT_PER, OUT_PER)])


@jax.jit
def kernel(x, W, b, emb):
    mesh = plsc.VectorSubcoreMesh(core_axis_name="c", subcore_axis_name="s")
    run = functools.partial(
        pl.kernel,
        mesh=mesh,
        out_type=jax.ShapeDtypeStruct((ROWS * 3,), jnp.float32),
        compiler_params=pltpu.CompilerParams(needs_layout_passes=False),
        scratch_types=[
            pltpu.VMEM((IN_PER,), jnp.float32),
            pltpu.VMEM((OUT_PER,), jnp.float32),
            pltpu.VMEM((128,), jnp.float32),
            pltpu.VMEM((128,), jnp.float32),
            pltpu.VMEM((128,), jnp.float32),
            pltpu.VMEM((128,), jnp.float32),
            pltpu.VMEM((128,), jnp.int32),
            pltpu.VMEM((128,), jnp.int32),
            pltpu.VMEM((128,), jnp.int32),
        ],
    )(_sc_body)
    out_flat = run(x.reshape(-1), W.reshape(-1), b, emb.reshape(-1),
                   jnp.asarray(_COL), jnp.asarray(_COL2), jnp.asarray(_ROW2))
    return out_flat.reshape(ROWS, 3)


# speedup vs baseline: 2.4167x; 2.4167x over previous
import functools

import jax
import jax.numpy as jnp
from jax import lax
from jax.experimental import pallas as pl
from jax.experimental.pallas import tpu as pltpu
from jax.experimental.pallas import tpu_sc as plsc

ROWS = 16384
NW = 32
NBLK = ROWS // 128
BLK_PER = NBLK // NW
IN_PER = BLK_PER * 256
OUT_PER = BLK_PER * 512

_W_OFF = 1
_B_OFF = 7
_EMB_OFF = 10


def _sc_body(x_hbm, p_hbm, out_hbm, xv, ov, pv):
    nc = 2
    wid = lax.axis_index("s") * nc + lax.axis_index("c")

    pltpu.sync_copy(x_hbm.at[pl.ds(wid * IN_PER, IN_PER)], xv)
    pltpu.sync_copy(p_hbm, pv.at[pl.ds(0, 48)])

    zero = lax.iota(jnp.int32, 16) * 0
    wa, wb, cc = [], [], []
    for c in range(3):
        wa.append(plsc.load_gather(pv, [zero + (_W_OFF + 2 * c)]))
        wb.append(plsc.load_gather(pv, [zero + (_W_OFF + 2 * c + 1)]))
        bc = plsc.load_gather(pv, [zero + (_B_OFF + c)])
        e0 = plsc.load_gather(pv, [zero + (_EMB_OFF + c)])
        cc.append(bc + e0)

    for kk in range(BLK_PER):
        for v in range(8):
            av = xv[pl.ds(256 * kk + 16 * v, 16)]
            bv = xv[pl.ds(256 * kk + 128 + 16 * v, 16)]
            for c in range(3):
                ov[pl.ds(512 * kk + 128 * c + 16 * v, 16)] = (
                    av * wa[c] + bv * wb[c] + cc[c]
                )

    pltpu.sync_copy(ov, out_hbm.at[pl.ds(wid * OUT_PER, OUT_PER)])


@jax.jit
def kernel(x, W, b, emb):
    x1d = x.reshape(NBLK, 128, 2).transpose(0, 2, 1).reshape(-1)
    params = jnp.concatenate(
        [jnp.zeros((1,), jnp.float32), W.reshape(-1), b, emb.reshape(-1), jnp.zeros((8,), jnp.float32)]
    )
    mesh = plsc.VectorSubcoreMesh(core_axis_name="c", subcore_axis_name="s")
    run = functools.partial(
        pl.kernel,
        mesh=mesh,
        out_type=jax.ShapeDtypeStruct((ROWS * 4,), jnp.float32),
        compiler_params=pltpu.CompilerParams(needs_layout_passes=False),
        scratch_types=[
            pltpu.VMEM((IN_PER,), jnp.float32),
            pltpu.VMEM((OUT_PER,), jnp.float32),
            pltpu.VMEM((128,), jnp.float32),
        ],
    )(_sc_body)
    out1d = run(x1d, params)
    return out1d.reshape(NBLK, 4, 128)[:, :3, :].transpose(0, 2, 1).reshape(ROWS, 3)
